# baseline (speedup 1.0000x reference)
import functools

import jax
import jax.numpy as jnp
from jax import lax
from jax.experimental import pallas as pl
from jax.experimental.pallas import tpu as pltpu
from jax.experimental.pallas import tpu_sc as plsc

B = 16384
D = 64
L = 16
NC = 2
NS = 16
NW = NC * NS
BPW = B // NW
CH = 128
NCH = BPW // CH


def _body(uidx_hbm, iidx_hbm, utab_hbm, itab_hbm, out_hbm,
          uidx_v, iidx_v, urows_v, irows_v, out_v, sem):
    wid = lax.axis_index("s") * NC + lax.axis_index("c")

    pltpu.sync_copy(uidx_hbm.at[wid], uidx_v)
    pltpu.sync_copy(iidx_hbm.at[wid], iidx_v)

    copies = []
    for k in range(NCH):
        copies.append(pltpu.async_copy(
            utab_hbm.at[uidx_v.at[k]], urows_v.at[pl.ds(k * CH, CH)], sem))
        copies.append(pltpu.async_copy(
            itab_hbm.at[iidx_v.at[k]], irows_v.at[pl.ds(k * CH, CH)], sem))
    for c in copies:
        c.wait()

    lanes = lax.iota(jnp.int32, L)

    def group(g, _):
        ridx = g * L + lanes

        def dstep(d, acc):
            cidx = jnp.full((L,), 0, jnp.int32) + d
            u = plsc.load_gather(urows_v, [ridx, cidx])
            v = plsc.load_gather(irows_v, [ridx, cidx])
            return acc + u * v

        acc = lax.fori_loop(0, D, dstep, jnp.zeros((L,), jnp.float32))
        out_v[pl.ds(g * L, L)] = acc
        return 0

    lax.fori_loop(0, BPW // L, group, 0)
    pltpu.sync_copy(out_v, out_hbm.at[pl.ds(wid * BPW, BPW)])


@functools.partial(
    pl.kernel,
    out_type=jax.ShapeDtypeStruct((B,), jnp.float32),
    mesh=plsc.VectorSubcoreMesh(core_axis_name="c", subcore_axis_name="s"),
    scratch_types=[
        pltpu.VMEM((NCH, CH), jnp.int32),
        pltpu.VMEM((NCH, CH), jnp.int32),
        pltpu.VMEM((BPW, D), jnp.float32),
        pltpu.VMEM((BPW, D), jnp.float32),
        pltpu.VMEM((BPW,), jnp.float32),
        pltpu.SemaphoreType.DMA,
    ],
    compiler_params=pltpu.CompilerParams(
        needs_layout_passes=False, use_tc_tiling_on_sc=False),
)
def _sc_dot(uidx_hbm, iidx_hbm, utab_hbm, itab_hbm, out_hbm, *scratch):
    _body(uidx_hbm, iidx_hbm, utab_hbm, itab_hbm, out_hbm, *scratch)


def kernel(user_idx, item_idx, user_table, item_table):
    uidx = user_idx.astype(jnp.int32).reshape(NW, NCH, CH)
    iidx = item_idx.astype(jnp.int32).reshape(NW, NCH, CH)
    return _sc_dot(uidx, iidx, user_table, item_table)


# speedup vs baseline: 1.5489x; 1.5489x over previous
import functools

import jax
import jax.numpy as jnp
from jax import lax
from jax.experimental import pallas as pl
from jax.experimental.pallas import tpu as pltpu
from jax.experimental.pallas import tpu_sc as plsc

B = 16384
D = 64
L = 16
NC = 2
NS = 16
NW = NC * NS
BPW = B // NW
CH = 128
NCH = BPW // CH


def _body(uidx_hbm, iidx_hbm, utab_hbm, itab_hbm, out_hbm,
          uidx_v, iidx_v, urows_v, irows_v, out_v, usem, isem):
    wid = lax.axis_index("s") * NC + lax.axis_index("c")
    base = wid * BPW

    pltpu.sync_copy(uidx_hbm.at[pl.ds(base, BPW)], uidx_v)
    pltpu.sync_copy(iidx_hbm.at[pl.ds(base, BPW)], iidx_v)

    lanes = lax.iota(jnp.int32, L)

    def fire_chunk(k):
        def issue(j, _):
            uvec = uidx_v[pl.ds(k * CH + j * L, L)]
            ivec = iidx_v[pl.ds(k * CH + j * L, L)]
            for lane in range(L):
                pltpu.async_copy(
                    utab_hbm.at[uvec[lane]], urows_v.at[j * L + lane], usem)
                pltpu.async_copy(
                    itab_hbm.at[ivec[lane]], irows_v.at[j * L + lane], isem)
            return 0
        lax.fori_loop(0, CH // L, issue, 0)

    def drain_chunk():
        def dwait(i, _):
            pltpu.make_async_copy(utab_hbm.at[0], urows_v.at[i], usem).wait()
            pltpu.make_async_copy(itab_hbm.at[0], irows_v.at[i], isem).wait()
            return 0
        lax.fori_loop(0, CH, dwait, 0)

    def compute_chunk(k):
        def group(g, _):
            ridx = g * L + lanes

            def dstep(d, acc):
                cidx = jnp.full((L,), 0, jnp.int32) + d
                u = plsc.load_gather(urows_v, [ridx, cidx])
                v = plsc.load_gather(irows_v, [ridx, cidx])
                return acc + u * v

            acc = lax.fori_loop(0, D, dstep, jnp.zeros((L,), jnp.float32))
            out_v[pl.ds(k * CH + g * L, L)] = acc
            return 0
        lax.fori_loop(0, CH // L, group, 0)

    for k in range(NCH):
        fire_chunk(k)
        drain_chunk()
        compute_chunk(k)

    pltpu.sync_copy(out_v, out_hbm.at[pl.ds(base, BPW)])


@functools.partial(
    pl.kernel,
    out_type=jax.ShapeDtypeStruct((B,), jnp.float32),
    mesh=plsc.VectorSubcoreMesh(core_axis_name="c", subcore_axis_name="s"),
    scratch_types=[
        pltpu.VMEM((BPW,), jnp.int32),
        pltpu.VMEM((BPW,), jnp.int32),
        pltpu.VMEM((CH, D), jnp.float32),
        pltpu.VMEM((CH, D), jnp.float32),
        pltpu.VMEM((BPW,), jnp.float32),
        pltpu.SemaphoreType.DMA,
        pltpu.SemaphoreType.DMA,
    ],
    compiler_params=pltpu.CompilerParams(needs_layout_passes=False),
)
def _sc_dot(uidx_hbm, iidx_hbm, utab_hbm, itab_hbm, out_hbm, *scratch):
    _body(uidx_hbm, iidx_hbm, utab_hbm, itab_hbm, out_hbm, *scratch)


def kernel(user_idx, item_idx, user_table, item_table):
    return _sc_dot(user_idx.astype(jnp.int32), item_idx.astype(jnp.int32),
                   user_table, item_table)
